# baseline (speedup 1.0000x reference)
import functools

import jax
import jax.numpy as jnp
from jax import lax
from jax.experimental import pallas as pl
from jax.experimental.pallas import tpu as pltpu
from jax.experimental.pallas import tpu_sc as plsc

_P = 0.12
_NC = 2
_NS = 16
_R = 4


def _jitter_indices(B, T):
    logits = jnp.log(jnp.array([_P / 2, 1 - _P, _P / 2], dtype=jnp.float32))
    key = jax.random.fold_in(jax.random.key(0), 1)
    idx = jax.random.categorical(key, logits, shape=(B, T)) - 1
    idx = idx.at[:, 0].set(jnp.clip(idx[:, 0], 0, 1))
    idx = idx.at[:, -1].set(jnp.clip(idx[:, -1], -1, 0))
    idx = idx + jnp.arange(T, dtype=idx.dtype)
    return idx.astype(jnp.int32)


def _sc_gather(B, C, T):
    L = 16
    R = 8
    SEG = T // 2
    HALO = 128
    W = SEG + HALO
    G = (C // R) * 2
    S = 3
    mesh = plsc.VectorSubcoreMesh(core_axis_name="c", subcore_axis_name="s")

    def body(x_hbm, idx_hbm, out_hbm, idx_v, *rest):
        bufs, sems = rest[:2 * S * R], rest[2 * S * R:]
        ins = tuple(bufs[i * R:(i + 1) * R] for i in range(S))
        outs = tuple(bufs[(S + i) * R:(S + i + 1) * R] for i in range(S))
        sin = sems[0:S]
        sout = sems[S:2 * S]

        b = lax.axis_index("s") * _NC + lax.axis_index("c")
        pltpu.sync_copy(idx_hbm.at[b], idx_v)

        def split(g):
            cb, h = g // 2, g % 2
            base = h * (SEG - HALO)
            return cb, h, base

        def in_copy(g, s, r):
            cb, _, base = split(g)
            return pltpu.make_async_copy(
                x_hbm.at[b, cb * R + r, pl.ds(base, W)], ins[s][r], sin[s])

        def out_copy(g, s, r):
            cb, h, _ = split(g)
            return pltpu.make_async_copy(
                outs[s][r], out_hbm.at[b, cb * R + r, pl.ds(h * SEG, SEG)],
                sout[s])

        for s in range(S):
            for r in range(R):
                in_copy(jnp.int32(s), s, r).start()

        def step(g, s):
            g = jnp.int32(g)
            _, h, base = split(g)
            toff = h * SEG
            for r in range(R):
                in_copy(g, s, r).wait()

            @pl.when(g >= S)
            def _wait_out():
                for r in range(R):
                    out_copy(g - S, s, r).wait()

            @plsc.parallel_loop(0, SEG, step=L, unroll=4)
            def _chunk(t0):
                src = idx_v[pl.ds(toff + t0, L)]
                loc = src - base
                for r in range(R):
                    v = plsc.load_gather(ins[s][r], [loc])
                    outs[s][r][pl.ds(t0, L)] = v
            for r in range(R):
                out_copy(g, s, r).start()

            @pl.when(g + S < G)
            def _next_in():
                for r in range(R):
                    in_copy(g + S, s, r).start()

        def group_block(i, carry):
            for s in range(S):
                step(i * S + s, s)
            return carry

        lax.fori_loop(0, G // S, group_block, 0)
        for g in range((G // S) * S, G):
            step(g, g % S)
        for g in range(G - S, G):
            for r in range(R):
                out_copy(jnp.int32(g), g % S, r).wait()

    return pl.kernel(
        body,
        out_type=jax.ShapeDtypeStruct((B, C, T), jnp.float32),
        mesh=mesh,
        compiler_params=pltpu.CompilerParams(needs_layout_passes=False),
        scratch_types=(
            [pltpu.VMEM((T,), jnp.int32)]
            + [pltpu.VMEM((W,), jnp.float32) for _ in range(S * R)]
            + [pltpu.VMEM((SEG,), jnp.float32) for _ in range(S * R)]
            + [pltpu.SemaphoreType.DMA for _ in range(2 * S)]
        ),
    )


@functools.lru_cache(maxsize=None)
def _build(B, C, T):
    return _sc_gather(B, C, T)


def kernel(x):
    B, C, T = x.shape
    idx = _jitter_indices(B, T)
    return _build(B, C, T)(x, idx)


# speedup vs baseline: 1.0126x; 1.0126x over previous
import functools

import jax
import jax.numpy as jnp
from jax import lax
from jax.experimental import pallas as pl
from jax.experimental.pallas import tpu as pltpu
from jax.experimental.pallas import tpu_sc as plsc

_P = 0.12
_NC = 2
_NS = 16
_R = 4


def _jitter_indices(B, T):
    logits = jnp.log(jnp.array([_P / 2, 1 - _P, _P / 2], dtype=jnp.float32))
    key = jax.random.fold_in(jax.random.key(0), 1)
    idx = jax.random.categorical(key, logits, shape=(B, T)) - 1
    idx = idx.at[:, 0].set(jnp.clip(idx[:, 0], 0, 1))
    idx = idx.at[:, -1].set(jnp.clip(idx[:, -1], -1, 0))
    idx = idx + jnp.arange(T, dtype=idx.dtype)
    return idx.astype(jnp.int32)


def _sc_gather(B, C, T):
    L = 16
    R = 4
    G = C // R
    S = 3
    mesh = plsc.VectorSubcoreMesh(core_axis_name="c", subcore_axis_name="s")

    def body(x_hbm, idx_hbm, out_hbm, idx_v, *rest):
        bufs, sems = rest[:2 * S * R], rest[2 * S * R:]
        ins = tuple(bufs[i * R:(i + 1) * R] for i in range(S))
        outs = tuple(bufs[(S + i) * R:(S + i + 1) * R] for i in range(S))
        sin = sems[0:S]
        sout = sems[S:2 * S]

        b = lax.axis_index("s") * _NC + lax.axis_index("c")
        pltpu.sync_copy(idx_hbm.at[b], idx_v)

        def in_copy(g, s, r):
            return pltpu.make_async_copy(
                x_hbm.at[b, g * R + r], ins[s][r], sin[s])

        def out_copy(g, s, r):
            return pltpu.make_async_copy(
                outs[s][r], out_hbm.at[b, g * R + r], sout[s])

        for s in range(S):
            for r in range(R):
                in_copy(jnp.int32(s), s, r).start()

        def step(g, s):
            g = jnp.int32(g)
            for r in range(R):
                in_copy(g, s, r).wait()

            @pl.when(g >= S)
            def _wait_out():
                for r in range(R):
                    out_copy(g - S, s, r).wait()

            @plsc.parallel_loop(0, T, step=L, unroll=4)
            def _chunk(t0):
                src = idx_v[pl.ds(t0, L)]
                for r in range(R):
                    v = plsc.load_gather(ins[s][r], [src])
                    outs[s][r][pl.ds(t0, L)] = v
            for r in range(R):
                out_copy(g, s, r).start()

            @pl.when(g + S < G)
            def _next_in():
                for r in range(R):
                    in_copy(g + S, s, r).start()

        def group_block(i, carry):
            for s in range(S):
                step(i * S + s, s)
            return carry

        lax.fori_loop(0, G // S, group_block, 0)
        for g in range((G // S) * S, G):
            step(g, g % S)
        for g in range(G - S, G):
            for r in range(R):
                out_copy(jnp.int32(g), g % S, r).wait()

    return pl.kernel(
        body,
        out_type=jax.ShapeDtypeStruct((B, C, T), jnp.float32),
        mesh=mesh,
        compiler_params=pltpu.CompilerParams(needs_layout_passes=False),
        scratch_types=(
            [pltpu.VMEM((T,), jnp.int32)]
            + [pltpu.VMEM((T,), jnp.float32) for _ in range(2 * S * R)]
            + [pltpu.SemaphoreType.DMA for _ in range(2 * S)]
        ),
    )


@functools.lru_cache(maxsize=None)
def _build(B, C, T):
    return _sc_gather(B, C, T)


def kernel(x):
    B, C, T = x.shape
    idx = _jitter_indices(B, T)
    return _build(B, C, T)(x, idx)


# speedup vs baseline: 1.0162x; 1.0035x over previous
import functools

import jax
import jax.numpy as jnp
from jax import lax
from jax.experimental import pallas as pl
from jax.experimental.pallas import tpu as pltpu
from jax.experimental.pallas import tpu_sc as plsc

_P = 0.12
_NC = 2


def _jitter_indices(B, T):
    logits = jnp.log(jnp.array([_P / 2, 1 - _P, _P / 2], dtype=jnp.float32))
    key = jax.random.fold_in(jax.random.key(0), 1)
    idx = jax.random.categorical(key, logits, shape=(B, T)) - 1
    idx = idx.at[:, 0].set(jnp.clip(idx[:, 0], 0, 1))
    idx = idx.at[:, -1].set(jnp.clip(idx[:, -1], -1, 0))
    idx = idx + jnp.arange(T, dtype=idx.dtype)
    return idx.astype(jnp.int32)


def _sc_gather(B, C, T):
    L = 16
    R = 4
    G = C // R
    S = 3
    mesh = plsc.VectorSubcoreMesh(core_axis_name="c", subcore_axis_name="s")

    def body(x_hbm, idx_hbm, out_hbm, idx_v, *rest):
        bufs, sems = rest[:2 * S * R], rest[2 * S * R:]
        ins = tuple(bufs[i * R:(i + 1) * R] for i in range(S))
        outs = tuple(bufs[(S + i) * R:(S + i + 1) * R] for i in range(S))
        sin = sems[0:S]
        sout = sems[S:2 * S]

        b = lax.axis_index("s") * _NC + lax.axis_index("c")
        pltpu.sync_copy(idx_hbm.at[b], idx_v)

        def in_copy(g, s, r):
            return pltpu.make_async_copy(
                x_hbm.at[b, g * R + r], ins[s][r], sin[s])

        def out_copy(g, s, r):
            return pltpu.make_async_copy(
                outs[s][r], out_hbm.at[b, g * R + r], sout[s])

        for s in range(S):
            for r in range(R):
                in_copy(jnp.int32(s), s, r).start()

        def step(g, s):
            g = jnp.int32(g)
            for r in range(R):
                in_copy(g, s, r).wait()

            @pl.when(g >= S)
            def _wait_out():
                for r in range(R):
                    out_copy(g - S, s, r).wait()

            @plsc.parallel_loop(0, T, step=L, unroll=4)
            def _chunk(t0):
                src = idx_v[pl.ds(t0, L)]
                for r in range(R):
                    v = plsc.load_gather(ins[s][r], [src])
                    outs[s][r][pl.ds(t0, L)] = v
            for r in range(R):
                out_copy(g, s, r).start()

            @pl.when(g + S < G)
            def _next_in():
                for r in range(R):
                    in_copy(g + S, s, r).start()

        def group_block(i, carry):
            for s in range(S):
                step(i * S + s, s)
            return carry

        lax.fori_loop(0, G // S, group_block, 0)
        for g in range((G // S) * S, G):
            step(g, g % S)
        for g in range(G - S, G):
            for r in range(R):
                out_copy(jnp.int32(g), g % S, r).wait()

    return pl.kernel(
        body,
        out_type=jax.ShapeDtypeStruct((B, C, T), jnp.float32),
        mesh=mesh,
        compiler_params=pltpu.CompilerParams(needs_layout_passes=False),
        scratch_types=(
            [pltpu.VMEM((T,), jnp.int32)]
            + [pltpu.VMEM((T,), jnp.float32) for _ in range(2 * S * R)]
            + [pltpu.SemaphoreType.DMA for _ in range(2 * S)]
        ),
    )


@functools.lru_cache(maxsize=None)
def _build(B, C, T):
    return _sc_gather(B, C, T)


def kernel(x):
    B, C, T = x.shape
    idx = _jitter_indices(B, T)
    return _build(B, C, T)(x, idx)
